# Initial kernel scaffold; baseline (speedup 1.0000x reference)
import jax
import jax.numpy as jnp
from jax.experimental import pallas as pl


def kernel(instance_feature, anchor, W):
    raise NotImplementedError("write your pallas kernel here")


# speedup vs baseline: 3.6076x; 3.6076x over previous
import functools

import jax
import jax.numpy as jnp
from jax import lax
from jax.experimental import pallas as pl
from jax.experimental.pallas import tpu as pltpu
from jax.experimental.pallas import tpu_sc as plsc

BB, GG, C = 4, 8192, 128
N = BB * GG
GXP = 131
PB = GXP * GXP
NCELL = BB * PB
HPAD = GXP + 1
BLK = 512
NOUT = 135 * BLK
NW = 32
SHARD = 2176
HG = NW * SHARD
NZ = 64
CHUNK = 128

_MESH = plsc.VectorSubcoreMesh(core_axis_name="c", subcore_axis_name="s")


def _prep_body(a_ref, o_ref):
    a = a_ref[...]
    xy = jax.nn.sigmoid(jnp.clip(a, -10.0, 10.0))
    mn = jnp.min(xy, axis=1, keepdims=True)
    gs = jnp.float32(1.0) / jnp.float32(128.0)
    ij = ((xy - mn) / gs).astype(jnp.int32)
    x = ij[0:1, :]
    y = ij[1:2, :]
    pid = lax.broadcasted_iota(jnp.int32, (1, N), 1)
    b = lax.shift_right_logical(pid, 13)
    o_ref[...] = b * PB + (x + 1) * GXP + (y + 1)


def _prep(a2):
    return pl.pallas_call(
        _prep_body,
        out_shape=jax.ShapeDtypeStruct((1, N), jnp.int32),
    )(a2)


def _grid_body(flat_hbm, feats_hbm, hg_hbm, flatbuf, map_v, rowbuf, sem):
    wid = lax.axis_index("s") * 2 + lax.axis_index("c")
    base = wid * SHARD
    pltpu.sync_copy(flat_hbm, flatbuf)

    lanes = lax.iota(jnp.int32, 16)

    def initb(i, carry):
        g = i * 16 + lanes
        vals = N + ((base + g) & (NZ - 1))
        plsc.store_scatter(map_v, [g], vals)
        return carry

    lax.fori_loop(0, SHARD // 16, initb, 0)

    def scanb(p, carry):
        loc = flatbuf[pl.ds(p * 16, 16)] + (HPAD - base)
        msk = (loc >= 0) & (loc < SHARD)
        locc = jnp.where(msk, loc, 0)
        idxs = p * 16 + lanes
        plsc.store_scatter(map_v, [locc], idxs, mask=msk)
        return carry

    lax.fori_loop(0, N // 16, scanb, 0)

    for cc in range(SHARD // CHUNK):
        pltpu.async_copy(
            feats_hbm.at[map_v.at[pl.ds(cc * CHUNK, CHUNK)]], rowbuf, sem).wait()
        pltpu.sync_copy(rowbuf, hg_hbm.at[pl.ds(base + cc * CHUNK, CHUNK)])


def _build_grid(flat, feats_ext):
    f = functools.partial(
        pl.kernel,
        out_type=jax.ShapeDtypeStruct((HG, C), jnp.float32),
        mesh=_MESH,
        compiler_params=pltpu.CompilerParams(needs_layout_passes=False),
        scratch_types=[
            pltpu.VMEM((N,), jnp.int32),
            pltpu.VMEM((SHARD,), jnp.int32),
            pltpu.VMEM((CHUNK, C), jnp.float32),
            pltpu.SemaphoreType.DMA,
        ],
    )(_grid_body)
    return f(flat, feats_ext)


def _conv_body(hg_hbm, w_ref, o_ref, hbuf, sem):
    i = pl.program_id(0)
    cp = pltpu.make_async_copy(
        hg_hbm.at[pl.ds(i * BLK, BLK + 2 * HPAD)], hbuf, sem)
    cp.start()
    cp.wait()
    acc = jnp.zeros((BLK, C), jnp.float32)
    for t in range(9):
        off = GXP * (t // 3) + (t % 3)
        acc = acc + jnp.dot(hbuf[pl.ds(off, BLK), :], w_ref[t],
                            preferred_element_type=jnp.float32)
    o_ref[...] = acc


def _conv(hgrid, wt):
    return pl.pallas_call(
        _conv_body,
        grid=(NOUT // BLK,),
        in_specs=[
            pl.BlockSpec(memory_space=pl.ANY),
            pl.BlockSpec((9, C, C), lambda i: (0, 0, 0)),
        ],
        out_specs=pl.BlockSpec((BLK, C), lambda i: (i, 0)),
        out_shape=jax.ShapeDtypeStruct((NOUT, C), jnp.float32),
        scratch_shapes=[
            pltpu.VMEM((BLK + 2 * HPAD, C), jnp.float32),
            pltpu.SemaphoreType.DMA,
        ],
    )(hgrid, wt)


def _out_body(og_hbm, flat_hbm, out_hbm, idxbuf, rowbuf, sem):
    wid = lax.axis_index("s") * 2 + lax.axis_index("c")
    ppw = N // NW
    pltpu.sync_copy(flat_hbm.at[pl.ds(wid * ppw, ppw)], idxbuf)
    for cc in range(ppw // CHUNK):
        pltpu.async_copy(
            og_hbm.at[idxbuf.at[pl.ds(cc * CHUNK, CHUNK)]], rowbuf, sem).wait()
        pltpu.sync_copy(rowbuf, out_hbm.at[pl.ds(wid * ppw + cc * CHUNK, CHUNK)])


def _gather_out(out_grid, flat):
    f = functools.partial(
        pl.kernel,
        out_type=jax.ShapeDtypeStruct((N, C), jnp.float32),
        mesh=_MESH,
        compiler_params=pltpu.CompilerParams(needs_layout_passes=False),
        scratch_types=[
            pltpu.VMEM((N // NW,), jnp.int32),
            pltpu.VMEM((CHUNK, C), jnp.float32),
            pltpu.SemaphoreType.DMA,
        ],
    )(_out_body)
    return f(out_grid, flat)


def kernel(instance_feature, anchor, W):
    feats = instance_feature.reshape(N, C).astype(jnp.float32)
    feats_ext = jnp.concatenate(
        [feats, jnp.zeros((NZ, C), jnp.float32)], axis=0)
    a2 = anchor[..., :2].reshape(N, 2).T
    wt = W.reshape(9, C, C)
    flat = _prep(a2).reshape(N)
    hgrid = _build_grid(flat, feats_ext)
    out_grid = _conv(hgrid, wt)
    out = _gather_out(out_grid, flat)
    return out.reshape(BB, GG, C)
